# baseline (speedup 1.0000x reference)
import jax
import jax.numpy as jnp
from jax.experimental import pallas as pl
from jax.experimental.pallas import tpu as pltpu

_HB = 32


def _body(t_ref, emb_ref, out_ref):
    row = emb_ref[0, pl.ds(t_ref[0] % 8, 1), :]
    out_ref[...] = jax.lax.broadcast_in_dim(row, out_ref.shape, (0, 3))


def kernel(x, t, embeddings):
    B, _, H, W = x.shape
    T, C = embeddings.shape
    t_arr = jnp.asarray(t, jnp.int32).reshape((1,))
    emb3 = embeddings.reshape(T // 8, 8, C)
    grid_spec = pltpu.PrefetchScalarGridSpec(
        num_scalar_prefetch=1,
        grid=(B, H // _HB),
        in_specs=[pl.BlockSpec((1, 8, C), lambda b, i, tr: (tr[0] // 8, 0, 0))],
        out_specs=pl.BlockSpec((1, _HB, W, C), lambda b, i, tr: (b, i, 0, 0)),
    )
    out = pl.pallas_call(
        _body,
        grid_spec=grid_spec,
        out_shape=jax.ShapeDtypeStruct((B, H, W, C), jnp.float32),
        compiler_params=pltpu.CompilerParams(
            dimension_semantics=("parallel", "parallel"),
        ),
    )(t_arr, emb3)
    return out.transpose(0, 3, 1, 2)
